# Initial kernel scaffold; baseline (speedup 1.0000x reference)
import jax
import jax.numpy as jnp
from jax.experimental import pallas as pl


def kernel(h, r, t, ent, rel):
    raise NotImplementedError("write your pallas kernel here")


# speedup vs baseline: 2.7169x; 2.7169x over previous
import functools

import jax
import jax.numpy as jnp
from jax import lax
from jax.experimental import pallas as pl
from jax.experimental.pallas import tpu as pltpu
from jax.experimental.pallas import tpu_sc as plsc

DIM = 64
BATCH = 16384
NC, NS, L = 2, 16, 16
NW = NC * NS
B_PER_W = BATCH // NW
C = 128
N_CHUNK = B_PER_W // C


def _trig_body(rel_ref, cs_ref):
    x = rel_ref[...]
    cs_ref[...] = jnp.concatenate([jnp.cos(x), jnp.sin(x)], axis=-1)


def _trig(rel):
    return pl.pallas_call(
        _trig_body,
        out_shape=jax.ShapeDtypeStruct((rel.shape[0], 2 * rel.shape[1]), rel.dtype),
    )(rel)


_mesh = plsc.VectorSubcoreMesh(
    core_axis_name="c", subcore_axis_name="s", num_cores=NC, num_subcores=NS
)


@functools.partial(
    pl.kernel,
    out_type=jax.ShapeDtypeStruct((BATCH,), jnp.float32),
    mesh=_mesh,
    scratch_types=[
        pltpu.VMEM((C,), jnp.int32),
        pltpu.VMEM((C,), jnp.int32),
        pltpu.VMEM((C,), jnp.int32),
        pltpu.VMEM((C, 2 * DIM), jnp.float32),
        pltpu.VMEM((C, 2 * DIM), jnp.float32),
        pltpu.VMEM((C, 2 * DIM), jnp.float32),
        pltpu.VMEM((C,), jnp.float32),
        pltpu.SemaphoreType.DMA,
    ],
)
def _sc_rotate(h_hbm, r_hbm, t_hbm, ent_hbm, cs_hbm, out_hbm,
               idx_h, idx_t, idx_r, hbuf, tbuf, csbuf, obuf, sem):
    wid = lax.axis_index("s") * NC + lax.axis_index("c")
    base = wid * B_PER_W

    def chunk_body(ci, carry):
        cbase = base + ci * C
        pltpu.sync_copy(h_hbm.at[pl.ds(cbase, C)], idx_h)
        pltpu.sync_copy(t_hbm.at[pl.ds(cbase, C)], idx_t)
        pltpu.sync_copy(r_hbm.at[pl.ds(cbase, C)], idx_r)
        cp1 = pltpu.async_copy(ent_hbm.at[idx_h], hbuf, sem)
        cp2 = pltpu.async_copy(ent_hbm.at[idx_t], tbuf, sem)
        cp3 = pltpu.async_copy(cs_hbm.at[idx_r], csbuf, sem)
        cp1.wait()
        cp2.wait()
        cp3.wait()

        lanes = lax.iota(jnp.int32, L)
        perms = [jnp.bitwise_xor(lanes, s) for s in (8, 4, 2, 1)]

        def block_body(b, carry2):
            res = jnp.zeros((L,), jnp.float32)
            for i2 in range(L):
                i = b * L + i2
                acc = jnp.zeros((L,), jnp.float32)
                for j in range(DIM // L):
                    lo = pl.ds(j * L, L)
                    hi_sl = pl.ds(DIM + j * L, L)
                    hr = hbuf[i, lo]
                    hi = hbuf[i, hi_sl]
                    tr = tbuf[i, lo]
                    ti = tbuf[i, hi_sl]
                    cz = csbuf[i, lo]
                    sz = csbuf[i, hi_sl]
                    rr = hr * cz - hi * sz - tr
                    ri = hr * sz + hi * cz - ti
                    acc = acc + jnp.abs(rr) + jnp.abs(ri)
                for p in perms:
                    acc = acc + jnp.take(acc, p)
                res = jnp.where(lanes == i2, -acc, res)
            obuf[pl.ds(b * L, L)] = res
            return carry2

        lax.fori_loop(0, C // L, block_body, 0)
        pltpu.sync_copy(obuf, out_hbm.at[pl.ds(cbase, C)])
        return carry

    lax.fori_loop(0, N_CHUNK, chunk_body, 0)


def kernel(h, r, t, ent, rel):
    cs = _trig(rel)
    return _sc_rotate(h, r, t, ent, cs)


# speedup vs baseline: 3.3148x; 1.2201x over previous
import functools

import jax
import jax.numpy as jnp
from jax import lax
from jax.experimental import pallas as pl
from jax.experimental.pallas import tpu as pltpu
from jax.experimental.pallas import tpu_sc as plsc

DIM = 64
BATCH = 16384
NC, NS, L = 2, 16, 16
NW = NC * NS
B_PER_W = BATCH // NW
C = 128
N_CHUNK = B_PER_W // C


def _trig_body(rel_ref, cs_ref):
    x = rel_ref[...]
    cs_ref[...] = jnp.concatenate([jnp.cos(x), jnp.sin(x)], axis=-1)


def _trig(rel):
    return pl.pallas_call(
        _trig_body,
        out_shape=jax.ShapeDtypeStruct((rel.shape[0], 2 * rel.shape[1]), rel.dtype),
    )(rel)


_mesh = plsc.VectorSubcoreMesh(
    core_axis_name="c", subcore_axis_name="s", num_cores=NC, num_subcores=NS
)


@functools.partial(
    pl.kernel,
    out_type=jax.ShapeDtypeStruct((BATCH,), jnp.float32),
    mesh=_mesh,
    scratch_types=[
        pltpu.VMEM((B_PER_W,), jnp.int32),
        pltpu.VMEM((B_PER_W,), jnp.int32),
        pltpu.VMEM((B_PER_W,), jnp.int32),
        pltpu.VMEM((2, C, 2 * DIM), jnp.float32),
        pltpu.VMEM((2, C, 2 * DIM), jnp.float32),
        pltpu.VMEM((2, C, 2 * DIM), jnp.float32),
        pltpu.VMEM((2, C), jnp.float32),
        pltpu.SemaphoreType.DMA,
        pltpu.SemaphoreType.DMA,
    ],
)
def _sc_rotate(h_hbm, r_hbm, t_hbm, ent_hbm, cs_hbm, out_hbm,
               idx_h, idx_t, idx_r, hbuf, tbuf, csbuf, obuf, sem0, sem1):
    wid = lax.axis_index("s") * NC + lax.axis_index("c")
    base = wid * B_PER_W
    sems = (sem0, sem1)

    pltpu.sync_copy(h_hbm.at[pl.ds(base, B_PER_W)], idx_h)
    pltpu.sync_copy(t_hbm.at[pl.ds(base, B_PER_W)], idx_t)
    pltpu.sync_copy(r_hbm.at[pl.ds(base, B_PER_W)], idx_r)

    def fire(ci, s):
        sem = sems[s]
        sl = pl.ds(ci * C, C)
        return (
            pltpu.async_copy(ent_hbm.at[idx_h.at[sl]], hbuf.at[s], sem),
            pltpu.async_copy(ent_hbm.at[idx_t.at[sl]], tbuf.at[s], sem),
            pltpu.async_copy(cs_hbm.at[idx_r.at[sl]], csbuf.at[s], sem),
        )

    lanes = lax.iota(jnp.int32, L)
    perms = [jnp.bitwise_xor(lanes, s) for s in (8, 4, 2, 1)]

    def compute(s):
        hb, tb, cb, ob = hbuf.at[s], tbuf.at[s], csbuf.at[s], obuf.at[s]

        def block_body(b, carry2):
            res = jnp.zeros((L,), jnp.float32)
            for i2 in range(L):
                i = b * L + i2
                acc = jnp.zeros((L,), jnp.float32)
                for j in range(DIM // L):
                    lo = pl.ds(j * L, L)
                    hi_sl = pl.ds(DIM + j * L, L)
                    hr = hb[i, lo]
                    hi = hb[i, hi_sl]
                    tr = tb[i, lo]
                    ti = tb[i, hi_sl]
                    cz = cb[i, lo]
                    sz = cb[i, hi_sl]
                    rr = hr * cz - hi * sz - tr
                    ri = hr * sz + hi * cz - ti
                    acc = acc + jnp.abs(rr) + jnp.abs(ri)
                for p in perms:
                    acc = acc + jnp.take(acc, p)
                res = jnp.where(lanes == i2, -acc, res)
            ob[pl.ds(b * L, L)] = res
            return carry2

        lax.fori_loop(0, C // L, block_body, 0)

    cps = fire(0, 0)
    for ci in range(N_CHUNK):
        s = ci % 2
        for cp in cps:
            cp.wait()
        if ci + 1 < N_CHUNK:
            cps = fire(ci + 1, 1 - s)
        compute(s)
        pltpu.sync_copy(obuf.at[s], out_hbm.at[pl.ds(base + ci * C, C)])


def kernel(h, r, t, ent, rel):
    cs = _trig(rel)
    return _sc_rotate(h, r, t, ent, cs)
